# baseline (speedup 1.0000x reference)
import functools

import jax
import jax.numpy as jnp
from jax import lax
from jax.experimental import pallas as pl
from jax.experimental.pallas import tpu as pltpu
from jax.experimental.pallas import tpu_sc as plsc

_K = 8192
_N = 16384
_LANES = 16
_NC = 2
_NS = 16
_NW = _NC * _NS
_QPW = _N // _NW
_NV = _QPW // _LANES
_UNROLL = 8
_RUNPROBE = 3

_mesh = plsc.VectorSubcoreMesh(core_axis_name="c", subcore_axis_name="s")


@functools.partial(
    pl.kernel,
    out_type=(
        jax.ShapeDtypeStruct((_N,), jnp.int32),
        jax.ShapeDtypeStruct((_N,), jnp.float32),
    ),
    mesh=_mesh,
    compiler_params=pltpu.CompilerParams(needs_layout_passes=False),
    scratch_types=[
        pltpu.VMEM((_K,), jnp.float32),
        pltpu.VMEM((_K,), jnp.int32),
        pltpu.VMEM((_QPW,), jnp.float32),
        pltpu.VMEM((_QPW,), jnp.int32),
        pltpu.VMEM((_QPW,), jnp.float32),
        pltpu.SemaphoreType.DMA,
        pltpu.SemaphoreType.DMA,
        pltpu.SemaphoreType.DMA,
    ],
)
def _vq_search(x_hbm, s_hbm, ord_hbm, oi_hbm, od_hbm,
               s_v, ord_v, q_v, oi_v, od_v, sem_a, sem_b, sem_c):
    wid = lax.axis_index("s") * _NC + lax.axis_index("c")
    base = wid * _QPW
    cp_s = pltpu.async_copy(s_hbm, s_v, sem_a)
    cp_o = pltpu.async_copy(ord_hbm, ord_v, sem_b)
    cp_q = pltpu.async_copy(x_hbm.at[pl.ds(base, _QPW)], q_v, sem_c)
    cp_s.wait()
    cp_o.wait()
    cp_q.wait()

    def runstart(p, v):
        for _ in range(_RUNPROBE):
            pm = jnp.maximum(p - 1, 0)
            vm = plsc.load_gather(s_v, [pm])
            p = jnp.where((p > 0) & (vm == v), pm, p)
        return p

    def splat(i):
        return plsc.load_gather(s_v, [jnp.full((_LANES,), i, jnp.int32)])

    t4095 = splat(4095)
    t2047, t6143 = splat(2047), splat(6143)
    t1023, t3071, t5119, t7167 = (splat(1023), splat(3071), splat(5119),
                                  splat(7167))
    (t511, t1535, t2559, t3583, t4607, t5631, t6655, t7679) = (
        splat(511), splat(1535), splat(2559), splat(3583), splat(4607),
        splat(5631), splat(6655), splat(7679))

    def descend4(x):
        w = jnp.where
        c0 = t4095 < x
        pos = w(c0, 4096, 0).astype(jnp.int32)
        c1 = w(c0, t6143, t2047) < x
        pos = pos + w(c1, 2048, 0)
        c2 = w(c0, w(c1, t7167, t5119), w(c1, t3071, t1023)) < x
        pos = pos + w(c2, 1024, 0)
        c3 = w(c0,
               w(c1, w(c2, t7679, t6655), w(c2, t5631, t4607)),
               w(c1, w(c2, t3583, t2559), w(c2, t1535, t511))) < x
        return pos + w(c3, 512, 0)

    for ci in range(_NV // _UNROLL):
        off = ci * (_UNROLL * _LANES)
        xs = [q_v[pl.ds(off + u * _LANES, _LANES)] for u in range(_UNROLL)]
        poss = [descend4(xs[u]) for u in range(_UNROLL)]
        step = _K // 32
        while step >= 1:
            for u in range(_UNROLL):
                sv = plsc.load_gather(s_v, [poss[u] + (step - 1)])
                poss[u] = poss[u] + jnp.where(sv < xs[u], step, 0)
            step //= 2
        for u in range(_UNROLL):
            pR = poss[u]
            pL = jnp.maximum(pR - 1, 0)
            vL = plsc.load_gather(s_v, [pL])
            vR = plsc.load_gather(s_v, [pR])
            iL = plsc.load_gather(ord_v, [runstart(pL, vL)])
            iR = plsc.load_gather(ord_v, [runstart(pR, vR)])
            dL = jnp.abs(xs[u] - vL)
            dR = jnp.abs(xs[u] - vR)
            takeR = (dR < dL) | ((dR == dL) & (iR < iL))
            oi_v[pl.ds(off + u * _LANES, _LANES)] = jnp.where(takeR, iR, iL)
            od_v[pl.ds(off + u * _LANES, _LANES)] = jnp.where(takeR, dR, dL)

    cp_oi = pltpu.async_copy(oi_v, oi_hbm.at[pl.ds(base, _QPW)], sem_a)
    cp_od = pltpu.async_copy(od_v, od_hbm.at[pl.ds(base, _QPW)], sem_b)
    cp_oi.wait()
    cp_od.wait()


def kernel(input_data, codewords):
    s, order = lax.sort(
        (codewords, jnp.arange(_K, dtype=jnp.int32)),
        num_keys=1, is_stable=True)
    return _vq_search(input_data, s, order)


# speedup vs baseline: 1.0168x; 1.0168x over previous
import functools

import jax
import jax.numpy as jnp
from jax import lax
from jax.experimental import pallas as pl
from jax.experimental.pallas import tpu as pltpu
from jax.experimental.pallas import tpu_sc as plsc

_K = 8192
_N = 16384
_LANES = 16
_NC = 2
_NS = 16
_NW = _NC * _NS
_QPW = _N // _NW
_NV = _QPW // _LANES
_UNROLL = 8
_RUNPROBE = 3

_mesh = plsc.VectorSubcoreMesh(core_axis_name="c", subcore_axis_name="s")


@functools.partial(
    pl.kernel,
    out_type=(
        jax.ShapeDtypeStruct((_N,), jnp.int32),
        jax.ShapeDtypeStruct((_N,), jnp.float32),
    ),
    mesh=_mesh,
    compiler_params=pltpu.CompilerParams(needs_layout_passes=False),
    scratch_types=[
        pltpu.VMEM((_K,), jnp.float32),
        pltpu.VMEM((_K,), jnp.int32),
        pltpu.VMEM((_QPW,), jnp.float32),
        pltpu.VMEM((_QPW,), jnp.int32),
        pltpu.VMEM((_QPW,), jnp.float32),
        pltpu.SemaphoreType.DMA,
        pltpu.SemaphoreType.DMA,
        pltpu.SemaphoreType.DMA,
    ],
)
def _vq_search(x_hbm, s_hbm, ord_hbm, oi_hbm, od_hbm,
               s_v, ord_v, q_v, oi_v, od_v, sem_a, sem_b, sem_c):
    wid = lax.axis_index("s") * _NC + lax.axis_index("c")
    base = wid * _QPW
    cp_s = pltpu.async_copy(s_hbm, s_v, sem_a)
    cp_o = pltpu.async_copy(ord_hbm, ord_v, sem_b)
    cp_q = pltpu.async_copy(x_hbm.at[pl.ds(base, _QPW)], q_v, sem_c)
    cp_s.wait()
    cp_o.wait()
    cp_q.wait()

    def runstart(p, v):
        for _ in range(_RUNPROBE):
            pm = jnp.maximum(p - 1, 0)
            vm = plsc.load_gather(s_v, [pm])
            p = jnp.where((p > 0) & (vm == v), pm, p)
        return p

    def chunk(ci, carry):
        off = ci * (_UNROLL * _LANES)
        xs = [q_v[pl.ds(off + u * _LANES, _LANES)] for u in range(_UNROLL)]
        poss = [jnp.zeros((_LANES,), jnp.int32) for _ in range(_UNROLL)]
        step = _K // 2
        while step >= 1:
            for u in range(_UNROLL):
                sv = plsc.load_gather(s_v, [poss[u] + (step - 1)])
                poss[u] = poss[u] + jnp.where(sv < xs[u], step, 0)
            step //= 2
        for u in range(_UNROLL):
            pR = poss[u]
            pL = jnp.maximum(pR - 1, 0)
            vL = plsc.load_gather(s_v, [pL])
            vR = plsc.load_gather(s_v, [pR])
            iL = plsc.load_gather(ord_v, [runstart(pL, vL)])
            iR = plsc.load_gather(ord_v, [runstart(pR, vR)])
            dL = jnp.abs(xs[u] - vL)
            dR = jnp.abs(xs[u] - vR)
            takeR = (dR < dL) | ((dR == dL) & (iR < iL))
            oi_v[pl.ds(off + u * _LANES, _LANES)] = jnp.where(takeR, iR, iL)
            od_v[pl.ds(off + u * _LANES, _LANES)] = jnp.where(takeR, dR, dL)
        return carry

    lax.fori_loop(0, _NV // _UNROLL, chunk, 0)
    cp_oi = pltpu.async_copy(oi_v, oi_hbm.at[pl.ds(base, _QPW)], sem_a)
    cp_od = pltpu.async_copy(od_v, od_hbm.at[pl.ds(base, _QPW)], sem_b)
    cp_oi.wait()
    cp_od.wait()


def kernel(input_data, codewords):
    s, order = lax.sort(
        (codewords, jnp.arange(_K, dtype=jnp.int32)),
        num_keys=1, is_stable=True)
    return _vq_search(input_data, s, order)
